# baseline (speedup 1.0000x reference)
import functools

import jax
import jax.numpy as jnp
from jax import lax
from jax.experimental import pallas as pl
from jax.experimental.pallas import tpu as pltpu
from jax.experimental.pallas import tpu_sc as plsc

B = 16384
D = 512
V = 32768
NB = 15
NC = 2
NS = 16
NW = NC * NS
BPW = B // NW
CH = 64
NCH = BPW // CH
L = 16


def _body(params_hbm, x_hbm, out_hbm, x_v, idx_v, buf0, buf1,
          gsem0, gsem1, osem0, osem1):
    wid = lax.axis_index("s") * NC + lax.axis_index("c")
    base = wid * BPW

    pltpu.sync_copy(x_hbm.at[:, pl.ds(base, BPW)], x_v)

    def jbody(j, _):
        for k in range(CH // L):
            off = j * CH + k * L
            acc = x_v[0, pl.ds(off, L)]
            for i in range(1, NB):
                acc = acc + acc + x_v[i, pl.ds(off, L)]
            idx_v[j, pl.ds(k * L, L)] = acc
        return 0
    lax.fori_loop(0, NCH, jbody, 0)

    bufs = (buf0, buf1)
    gsems = (gsem0, gsem1)
    osems = (osem0, osem1)

    def gather_start(j):
        b = j & 1
        pltpu.async_copy(params_hbm.at[idx_v.at[j]], bufs[b], gsems[b])

    def gather_wait(j):
        b = j & 1
        pltpu.make_async_copy(params_hbm.at[idx_v.at[j]], bufs[b],
                              gsems[b]).wait()

    def out_start(j):
        b = j & 1
        pltpu.async_copy(bufs[b], out_hbm.at[pl.ds(base + j * CH, CH)],
                         osems[b])

    def out_wait(j):
        b = j & 1
        pltpu.make_async_copy(bufs[b], out_hbm.at[pl.ds(base + j * CH, CH)],
                              osems[b]).wait()

    gather_start(0)
    for j in range(NCH):
        if j + 1 < NCH:
            if j - 1 >= 0:
                out_wait(j - 1)
            gather_start(j + 1)
        gather_wait(j)
        out_start(j)
    out_wait(NCH - 2)
    out_wait(NCH - 1)


_gather = functools.partial(
    pl.kernel,
    mesh=plsc.VectorSubcoreMesh(core_axis_name="c", subcore_axis_name="s"),
    out_type=jax.ShapeDtypeStruct((B, D), jnp.float32),
    scratch_types=[
        pltpu.VMEM((NB, BPW), jnp.int32),
        pltpu.VMEM((NCH, CH), jnp.int32),
        pltpu.VMEM((CH, D), jnp.float32),
        pltpu.VMEM((CH, D), jnp.float32),
        pltpu.SemaphoreType.DMA,
        pltpu.SemaphoreType.DMA,
        pltpu.SemaphoreType.DMA,
        pltpu.SemaphoreType.DMA,
    ],
)(_body)


@jax.jit
def kernel(x, params):
    table = params.reshape(V, D)
    out = _gather(table, x)
    return out.reshape(B, 8, 64)
